# baseline (speedup 1.0000x reference)
import functools

import jax
import jax.numpy as jnp
from jax import lax
from jax.experimental import pallas as pl
from jax.experimental.pallas import tpu as pltpu
from jax.experimental.pallas import tpu_sc as plsc

NUM_NODES = 1000000
EMBED_DIM = 64
BATCH = 16384
SC_ROWS = 11264
TC_ROWS = BATCH - SC_ROWS

_info = plsc.get_sparse_core_info()
_NC, _NS, _L = _info.num_cores, _info.num_subcores, _info.num_lanes
_NW = _NC * _NS
_B_PER_W = SC_ROWS // _NW


@functools.partial(
    pl.kernel,
    mesh=plsc.VectorSubcoreMesh(core_axis_name="c", subcore_axis_name="s"),
    out_type=jax.ShapeDtypeStruct((SC_ROWS, EMBED_DIM), jnp.float32),
    scratch_types=[
        pltpu.VMEM((_B_PER_W,), jnp.int32),
        pltpu.VMEM((_B_PER_W, EMBED_DIM), jnp.float32),
        pltpu.SemaphoreType.DMA,
    ],
)
def _gather_sc(table_hbm, idx_hbm, out_hbm, idx_v, rows_v, sem):
    wid = lax.axis_index("s") * _NC + lax.axis_index("c")
    base = wid * _B_PER_W
    pltpu.sync_copy(idx_hbm.at[pl.ds(base, _B_PER_W)], idx_v)

    def fire(g, carry):
        vec = idx_v[pl.ds(g * _L, _L)]
        for t in range(_L):
            i = vec[t]
            pltpu.make_async_copy(
                table_hbm.at[i], rows_v.at[g * _L + t], sem
            ).start()
        return carry

    lax.fori_loop(0, _B_PER_W // _L, fire, 0)
    pltpu.make_async_copy(
        table_hbm.at[pl.ds(0, _B_PER_W)], rows_v, sem
    ).wait()
    pltpu.sync_copy(rows_v, out_hbm.at[pl.ds(base, _B_PER_W)])


def _tc_body(idx_ref, *refs):
    *in_refs, out_ref = refs
    i = pl.program_id(0)
    for t, in_ref in enumerate(in_refs):
        m = lax.rem(idx_ref[8 * i + t], 8)
        out_ref[pl.ds(t, 1), :] = in_ref[pl.ds(m, 1), :]


_gather_tc = pl.pallas_call(
    _tc_body,
    grid_spec=pltpu.PrefetchScalarGridSpec(
        num_scalar_prefetch=1,
        grid=(TC_ROWS // 8,),
        in_specs=[
            pl.BlockSpec(
                (8, EMBED_DIM),
                (lambda i, idx_ref, t=t: (idx_ref[8 * i + t] // 8, 0)),
            )
            for t in range(8)
        ],
        out_specs=pl.BlockSpec((8, EMBED_DIM), lambda i, idx_ref: (i, 0)),
    ),
    out_shape=jax.ShapeDtypeStruct((TC_ROWS, EMBED_DIM), jnp.float32),
)


def kernel(node_indices, embeddings):
    idx = node_indices.astype(jnp.int32)
    out_sc = _gather_sc(embeddings, idx[:SC_ROWS])
    out_tc = _gather_tc(idx[SC_ROWS:], *([embeddings] * 8))
    return jnp.concatenate([out_sc, out_tc], axis=0)


# speedup vs baseline: 1.9138x; 1.9138x over previous
import functools

import jax
import jax.numpy as jnp
from jax import lax
from jax.experimental import pallas as pl
from jax.experimental.pallas import tpu as pltpu
from jax.experimental.pallas import tpu_sc as plsc

NUM_NODES = 1000000
EMBED_DIM = 64
BATCH = 16384

_info = plsc.get_sparse_core_info()
_NC, _NS, _L = _info.num_cores, _info.num_subcores, _info.num_lanes
_NW = _NC * _NS
_B_PER_W = BATCH // _NW


@functools.partial(
    pl.kernel,
    mesh=plsc.VectorSubcoreMesh(core_axis_name="c", subcore_axis_name="s"),
    out_type=jax.ShapeDtypeStruct((BATCH, EMBED_DIM), jnp.float32),
    scratch_types=[
        pltpu.VMEM((_B_PER_W,), jnp.int32),
        pltpu.VMEM((_B_PER_W, EMBED_DIM), jnp.float32),
        pltpu.SemaphoreType.DMA,
    ],
)
def _gather_kernel(table_hbm, idx_hbm, out_hbm, idx_v, rows_v, sem):
    wid = lax.axis_index("s") * _NC + lax.axis_index("c")
    base = wid * _B_PER_W
    pltpu.sync_copy(idx_hbm.at[pl.ds(base, _B_PER_W)], idx_v)

    def fire(g, carry):
        vec = idx_v[pl.ds(g * _L, _L)]
        for t in range(_L):
            i = vec[t]
            pltpu.make_async_copy(
                table_hbm.at[i], rows_v.at[g * _L + t], sem
            ).start()
        return carry

    lax.fori_loop(0, _B_PER_W // _L, fire, 0)
    pltpu.make_async_copy(
        table_hbm.at[pl.ds(0, _B_PER_W)], rows_v, sem
    ).wait()
    pltpu.sync_copy(rows_v, out_hbm.at[pl.ds(base, _B_PER_W)])


def kernel(node_indices, embeddings):
    idx = node_indices.astype(jnp.int32)
    return _gather_kernel(embeddings, idx)
